# Initial kernel scaffold; baseline (speedup 1.0000x reference)
import jax
import jax.numpy as jnp
from jax.experimental import pallas as pl


def kernel(x, edge_index, edge_weights, W, b):
    raise NotImplementedError("write your pallas kernel here")


# speedup vs baseline: 25.2177x; 25.2177x over previous
import functools

import jax
import jax.numpy as jnp
from jax import lax
from jax.experimental import pallas as pl
from jax.experimental.pallas import tpu as pltpu
from jax.experimental.pallas import tpu_sc as plsc

NC = 2
NS = 16
NW = NC * NS
L = 16
CB = 128
ZB = 2048


def _edge_partition(e):
    tot_ch = e // CB
    units = tot_ch // 8
    rem = tot_ch % 8
    u_lo = units // NW
    u_x = units % NW
    rows_hi, rows_lo = 8 * (u_lo + 1), 8 * u_lo
    nch = max(rows_hi, rows_lo + rem)
    return rows_hi, rows_lo, u_x, rem, nch


def _stage_edges(e, wid, hbm2d_refs, vmem2d_refs, w_hbm, wbuf):
    rows_hi, rows_lo, u_x, rem, _ = _edge_partition(e)
    units8 = (e // CB) - rem

    @pl.when(wid < u_x)
    def _():
        base = pl.multiple_of(rows_hi * wid, 8)
        for h, v in zip(hbm2d_refs, vmem2d_refs):
            pltpu.sync_copy(h.at[pl.ds(base, rows_hi)],
                            v.at[pl.ds(0, rows_hi)])
        pltpu.sync_copy(w_hbm.at[pl.ds(pl.multiple_of(base * CB, 8),
                                       rows_hi * CB)],
                        wbuf.at[pl.ds(0, rows_hi * CB)])

    @pl.when(wid >= u_x)
    def _():
        base = pl.multiple_of(rows_lo * wid + 8 * u_x, 8)
        for h, v in zip(hbm2d_refs, vmem2d_refs):
            pltpu.sync_copy(h.at[pl.ds(base, rows_lo)],
                            v.at[pl.ds(0, rows_lo)])
        pltpu.sync_copy(w_hbm.at[pl.ds(pl.multiple_of(base * CB, 8),
                                       rows_lo * CB)],
                        wbuf.at[pl.ds(0, rows_lo * CB)])

    if rem:
        @pl.when(wid == NW - 1)
        def _():
            for h, v in zip(hbm2d_refs, vmem2d_refs):
                pltpu.sync_copy(h.at[pl.ds(units8, rem)],
                                v.at[pl.ds(rows_lo, rem)])
            pltpu.sync_copy(w_hbm.at[pl.ds(units8 * CB, rem * CB)],
                            wbuf.at[pl.ds(rows_lo * CB, rem * CB)])

    my_n = jnp.where(wid < u_x, rows_hi, rows_lo)
    if rem:
        my_n = my_n + jnp.where(wid == NW - 1, rem, 0)
    return my_n


def _deg_body(n_pad, e, dst_hbm, w_hbm, out_hbm, dstb, wbuf, zbuf, deg_sh):
    cid = lax.axis_index("c")
    sid = lax.axis_index("s")
    wid = sid * NC + cid

    @pl.when(sid == 0)
    def _():
        def zb(i, c):
            zbuf[pl.ds(i * L, L)] = jnp.zeros((L,), jnp.float32)
            return c
        lax.fori_loop(0, ZB // L, zb, 0)
        for k in range(n_pad // ZB):
            pltpu.sync_copy(zbuf, deg_sh.at[pl.ds(k * ZB, ZB)])

    my_n = _stage_edges(e, wid, [dst_hbm], [dstb], w_hbm, wbuf)

    _, _, _, _, nch = _edge_partition(e)

    def clipb(i, c):
        wbuf[pl.ds(i * L, L)] = jnp.clip(wbuf[pl.ds(i * L, L)], -2.0, 5.0)
        return c
    lax.fori_loop(0, nch * CB // L, clipb, 0)

    plsc.subcore_barrier()

    def chunk(j, c):
        pltpu.sync_copy(wbuf.at[pl.ds(j * CB, CB)],
                        deg_sh.at[dstb.at[j]], add=True)
        return c
    lax.fori_loop(0, my_n, chunk, 0)

    plsc.subcore_barrier()

    @pl.when(sid == 0)
    def _():
        for k in range(n_pad // ZB):
            pltpu.sync_copy(deg_sh.at[pl.ds(k * ZB, ZB)], zbuf)
            pltpu.sync_copy(zbuf, out_hbm.at[pl.ds(
                pl.multiple_of(cid * n_pad + k * ZB, 8), ZB)])


@functools.lru_cache(None)
def _deg_sc(n_pad, e):
    nch = _edge_partition(e)[4]
    mesh = plsc.VectorSubcoreMesh(core_axis_name="c", subcore_axis_name="s")
    return pl.kernel(
        functools.partial(_deg_body, n_pad, e),
        out_type=jax.ShapeDtypeStruct((NC * n_pad,), jnp.float32),
        mesh=mesh,
        scratch_types=[
            pltpu.VMEM((nch, CB), jnp.int32),
            pltpu.VMEM((nch * CB,), jnp.float32),
            pltpu.VMEM((ZB,), jnp.float32),
            pltpu.VMEM_SHARED((n_pad,), jnp.float32),
        ],
    )


def _stripe_copy(n, sid, fn):
    full = n // CB
    tail = n - full * CB
    for t in range(full // NS):
        fn(pl.multiple_of((sid + NS * t) * CB, 8), CB)
    last_t = full // NS
    rem_full = full % NS

    @pl.when(sid < rem_full)
    def _():
        fn(pl.multiple_of((sid + NS * last_t) * CB, 8), CB)

    if tail:
        @pl.when(sid == rem_full)
        def _():
            fn(full * CB, tail)


def _agg_body(n, e, d, src_hbm, dst_hbm, w_hbm, y2_hbm, out_hbm,
              srcb, dstb, wbuf, rowbuf, s_sh, sem):
    cid = lax.axis_index("c")
    sid = lax.axis_index("s")
    wid = sid * NC + cid

    my_n = _stage_edges(e, wid, [src_hbm, dst_hbm], [srcb, dstb],
                        w_hbm, wbuf)

    def zrow(j, c):
        for g in range(d // L):
            rowbuf[j, pl.ds(g * L, L)] = jnp.zeros((L,), jnp.float32)
        return c
    lax.fori_loop(0, CB, zrow, 0)

    _stripe_copy(n, sid, lambda r0, sz: pltpu.sync_copy(
        rowbuf.at[pl.ds(0, sz)], s_sh.at[pl.ds(r0, sz)]))
    plsc.subcore_barrier()

    def chunk(j, c):
        pltpu.async_copy(y2_hbm.at[srcb.at[j]], rowbuf, sem).wait()

        def group(gi, c2):
            wgrp = jnp.clip(wbuf[pl.ds(j * CB + gi * L, L)], -2.0, 5.0)
            for lane in range(L):
                wv = wgrp[lane]
                ei = gi * L + lane
                for g in range(d // L):
                    rowbuf[ei, pl.ds(g * L, L)] = (
                        rowbuf[ei, pl.ds(g * L, L)] * wv)
            return c2
        lax.fori_loop(0, CB // L, group, 0)

        pltpu.sync_copy(rowbuf, s_sh.at[dstb.at[j]], add=True)
        return c
    lax.fori_loop(0, my_n, chunk, 0)

    plsc.subcore_barrier()

    _stripe_copy(n, sid, lambda r0, sz: pltpu.sync_copy(
        s_sh.at[pl.ds(r0, sz)], out_hbm.at[cid, pl.ds(r0, sz)]))


@functools.lru_cache(None)
def _agg_sc(n, e, d):
    nch = _edge_partition(e)[4]
    mesh = plsc.VectorSubcoreMesh(core_axis_name="c", subcore_axis_name="s")
    return pl.kernel(
        functools.partial(_agg_body, n, e, d),
        out_type=jax.ShapeDtypeStruct((NC, n, d), jnp.float32),
        mesh=mesh,
        scratch_types=[
            pltpu.VMEM((nch, CB), jnp.int32),
            pltpu.VMEM((nch, CB), jnp.int32),
            pltpu.VMEM((nch * CB,), jnp.float32),
            pltpu.VMEM((CB, d), jnp.float32),
            pltpu.VMEM_SHARED((n, d), jnp.float32),
            pltpu.SemaphoreType.DMA,
        ],
    )


def _linear_tc(x, w):
    n, d_in = x.shape
    d_out = w.shape[0]
    blk = 512

    def body(x_ref, w_ref, o_ref):
        o_ref[...] = lax.dot_general(
            x_ref[...], w_ref[...], (((1,), (1,)), ((), ())),
            preferred_element_type=jnp.float32,
            precision=lax.Precision.HIGHEST)

    return pl.pallas_call(
        body,
        grid=(pl.cdiv(n, blk),),
        in_specs=[pl.BlockSpec((blk, d_in), lambda i: (i, 0)),
                  pl.BlockSpec((d_out, d_in), lambda i: (0, 0))],
        out_specs=pl.BlockSpec((blk, d_out), lambda i: (i, 0)),
        out_shape=jax.ShapeDtypeStruct((n, d_out), jnp.float32),
    )(x, w)


def _prep_tc(pdeg, y):
    nc, n = pdeg.shape
    d = y.shape[1]
    blk = 512

    def body(pd_ref, y_ref, o_ref):
        deg = jnp.sum(pd_ref[...], axis=0) + 1.0
        dis = jnp.where(deg > 0.0, lax.rsqrt(deg), 0.0)
        o_ref[...] = y_ref[...] * dis[:, None]

    return pl.pallas_call(
        body,
        grid=(pl.cdiv(n, blk),),
        in_specs=[pl.BlockSpec((nc, blk), lambda i: (0, i)),
                  pl.BlockSpec((blk, d), lambda i: (i, 0))],
        out_specs=pl.BlockSpec((blk, d), lambda i: (i, 0)),
        out_shape=jax.ShapeDtypeStruct((n, d), jnp.float32),
    )(pdeg, y)


def _finalize_tc(pdeg, y2, s, b2):
    nc, n = pdeg.shape
    d = y2.shape[1]
    blk = 512

    def body(pd_ref, y2_ref, s_ref, b_ref, o_ref):
        deg = jnp.sum(pd_ref[...], axis=0) + 1.0
        dis = jnp.where(deg > 0.0, lax.rsqrt(deg), 0.0)
        acc = s_ref[0] + s_ref[1] + y2_ref[...]
        o_ref[...] = acc * dis[:, None] + b_ref[...]

    return pl.pallas_call(
        body,
        grid=(pl.cdiv(n, blk),),
        in_specs=[pl.BlockSpec((nc, blk), lambda i: (0, i)),
                  pl.BlockSpec((blk, d), lambda i: (i, 0)),
                  pl.BlockSpec((2, blk, d), lambda i: (0, i, 0)),
                  pl.BlockSpec((1, d), lambda i: (0, 0))],
        out_specs=pl.BlockSpec((blk, d), lambda i: (i, 0)),
        out_shape=jax.ShapeDtypeStruct((n, d), jnp.float32),
    )(pdeg, y2, s, b2)


def kernel(x, edge_index, edge_weights, W, b):
    n, d_in = x.shape
    e = edge_weights.shape[0]
    d_out = W.shape[0]
    assert e % CB == 0

    src2 = edge_index[0].reshape(e // CB, CB)
    dst2 = edge_index[1].reshape(e // CB, CB)

    n_pad = -(-n // ZB) * ZB
    pdeg = _deg_sc(n_pad, e)(dst2, edge_weights)
    pdeg = pdeg.reshape(NC, n_pad)[:, :n]
    y = _linear_tc(x, W)
    y2 = _prep_tc(pdeg, y)
    s = _agg_sc(n, e, d_out)(src2, dst2, edge_weights, y2)
    return _finalize_tc(pdeg, y2, s, b.reshape(1, d_out))
